# baseline (speedup 1.0000x reference)
import jax
import jax.numpy as jnp
from jax.experimental import pallas as pl

PAD = 0.0


def _pad_mask_kernel(x_ref, out_ref, mask_ref):
    x = x_ref[...]
    out_ref[...] = x
    mask_ref[...] = (x == PAD).astype(jnp.int8)


def kernel(tokens_batch):
    B, L = tokens_batch.shape
    nsteps = 4
    blk = L // nsteps
    out, mask8 = pl.pallas_call(
        _pad_mask_kernel,
        grid=(nsteps,),
        in_specs=[pl.BlockSpec((B, blk), lambda i: (0, i))],
        out_specs=(
            pl.BlockSpec((B, blk), lambda i: (0, i)),
            pl.BlockSpec((B, blk), lambda i: (0, i)),
        ),
        out_shape=(
            jax.ShapeDtypeStruct((B, L), tokens_batch.dtype),
            jax.ShapeDtypeStruct((B, L), jnp.int8),
        ),
    )(tokens_batch)
    return (out, mask8.astype(jnp.bool_))


# speedup vs baseline: 1.3497x; 1.3497x over previous
import jax
import jax.numpy as jnp
from jax.experimental import pallas as pl

PAD = 0.0


def _pad_mask_kernel(x_ref, out_ref, mask_ref):
    x = x_ref[...]
    out_ref[...] = x
    mask_ref[...] = (x == PAD).astype(jnp.int8)


def kernel(tokens_batch):
    B, L = tokens_batch.shape
    nsteps = 2
    blk = L // nsteps
    out, mask8 = pl.pallas_call(
        _pad_mask_kernel,
        grid=(nsteps,),
        in_specs=[pl.BlockSpec((B, blk), lambda i: (0, i))],
        out_specs=(
            pl.BlockSpec((B, blk), lambda i: (0, i)),
            pl.BlockSpec((B, blk), lambda i: (0, i)),
        ),
        out_shape=(
            jax.ShapeDtypeStruct((B, L), tokens_batch.dtype),
            jax.ShapeDtypeStruct((B, L), jnp.int8),
        ),
    )(tokens_batch)
    return (out, mask8.astype(jnp.bool_))
